# baseline (speedup 1.0000x reference)
import jax
import jax.numpy as jnp
from jax import lax
from jax.experimental import pallas as pl
from jax.experimental.pallas import tpu as pltpu
from jax.experimental.pallas import tpu_sc as plsc
import functools

N = 10000
D = 128
H = 128
E = 320000

NC = 2
NS = 16
NW = NC * NS
C = 128
CH = -(-E // (NW * C))
EPT = CH * C
E_PAD = NW * EPT
RPT = 632
N_ACC = NS * RPT

_MESH = plsc.VectorSubcoreMesh(
    core_axis_name="c", subcore_axis_name="s", num_cores=NC, num_subcores=NS)


@functools.partial(
    pl.kernel,
    out_type=[
        jax.ShapeDtypeStruct((NC, N_ACC, H), jnp.float32),
        jax.ShapeDtypeStruct((NW * N_ACC,), jnp.float32),
    ],
    mesh=_MESH,
    compiler_params=pltpu.CompilerParams(needs_layout_passes=False),
    scratch_types=[
        pltpu.VMEM((2, C), jnp.int32),
        pltpu.VMEM((C, H), jnp.float32),
        pltpu.VMEM((C,), jnp.float32),
        pltpu.VMEM((N_ACC,), jnp.float32),
        pltpu.VMEM_SHARED((N_ACC, H), jnp.float32),
        pltpu.SemaphoreType.DMA,
    ],
)
def _agg_cnt_kernel(h_hbm, idx_hbm, z2_hbm, z1_hbm,
                    sums_out, cnts_out,
                    ij_v, rows_v, ones_v, cnt_v, acc_sh,
                    sem):
    c = lax.axis_index("c")
    s = lax.axis_index("s")
    w = c * NS + s
    r0 = s * RPT
    pltpu.sync_copy(z2_hbm.at[pl.ds(r0, RPT)], acc_sh.at[pl.ds(r0, RPT)])
    pltpu.sync_copy(z1_hbm, cnt_v)
    one16 = jnp.ones((16,), jnp.float32)
    for i in range(C // 16):
        ones_v[pl.ds(i * 16, 16)] = one16
    plsc.subcore_barrier()

    def chunk(j, carry):
        pltpu.sync_copy(idx_hbm.at[w, j], ij_v)
        pltpu.async_copy(h_hbm.at[ij_v.at[0]], rows_v, sem).wait()
        pltpu.sync_copy(rows_v, acc_sh.at[ij_v.at[1]], add=True)
        for l in range(C // 16):
            d16 = ij_v[1, pl.ds(l * 16, 16)]
            plsc.addupdate_scatter(cnt_v, [d16], one16)
        return carry

    lax.fori_loop(0, CH, chunk, 0)
    plsc.subcore_barrier()
    pltpu.sync_copy(acc_sh.at[pl.ds(r0, RPT)], sums_out.at[c, pl.ds(r0, RPT)])
    pltpu.sync_copy(cnt_v, cnts_out.at[pl.ds(w * N_ACC, N_ACC)])


@functools.partial(
    pl.kernel,
    out_type=[jax.ShapeDtypeStruct((NC, N_ACC, H), jnp.float32)],
    mesh=_MESH,
    scratch_types=[
        pltpu.VMEM((2, C), jnp.int32),
        pltpu.VMEM((C, H), jnp.float32),
        pltpu.VMEM_SHARED((N_ACC, H), jnp.float32),
        pltpu.SemaphoreType.DMA,
    ],
)
def _agg_kernel(h_hbm, idx_hbm, z2_hbm,
                sums_out, ij_v, rows_v, acc_sh, sem):
    c = lax.axis_index("c")
    s = lax.axis_index("s")
    w = c * NS + s
    r0 = s * RPT
    pltpu.sync_copy(z2_hbm.at[pl.ds(r0, RPT)], acc_sh.at[pl.ds(r0, RPT)])
    plsc.subcore_barrier()

    def chunk(j, carry):
        pltpu.sync_copy(idx_hbm.at[w, j], ij_v)
        pltpu.async_copy(h_hbm.at[ij_v.at[0]], rows_v, sem).wait()
        pltpu.sync_copy(rows_v, acc_sh.at[ij_v.at[1]], add=True)
        return carry

    lax.fori_loop(0, CH, chunk, 0)
    plsc.subcore_barrier()
    pltpu.sync_copy(acc_sh.at[pl.ds(r0, RPT)], sums_out.at[c, pl.ds(r0, RPT)])


BR = 400
GRID = (N // BR,)


def _dotT(a, w):
    return lax.dot_general(a, w, (((1,), (1,)), ((), ())),
                           preferred_element_type=jnp.float32)


def _proj_body(x_ref, w_ref, b_ref, h0_ref, h_ref):
    t = _dotT(x_ref[...], w_ref[...]) + b_ref[...]
    h0_ref[...] = t
    h_ref[...] = jnp.maximum(t, 0.0)


def _proj(x, w1, b1):
    return pl.pallas_call(
        _proj_body,
        grid=GRID,
        in_specs=[
            pl.BlockSpec((BR, D), lambda i: (i, 0)),
            pl.BlockSpec((H, D), lambda i: (0, 0)),
            pl.BlockSpec((1, H), lambda i: (0, 0)),
        ],
        out_specs=[pl.BlockSpec((BR, H), lambda i: (i, 0))] * 2,
        out_shape=[jax.ShapeDtypeStruct((N, H), jnp.float32)] * 2,
    )(x, w1, b1.reshape(1, H))


def _mix0_body(s_ref, c_ref, h_ref, wl_ref, bl_ref, wr_ref, o_ref):
    cnt = jnp.sum(c_ref[...], axis=0)
    inv = 1.0 / jnp.maximum(cnt, 1.0)
    mean = (s_ref[0] + s_ref[1]) * inv
    o = _dotT(mean, wl_ref[...]) + bl_ref[...] + _dotT(h_ref[...], wr_ref[...])
    o_ref[...] = jnp.maximum(o, 0.0)


def _mix0(sums, cnts3, h, wl, bl, wr):
    return pl.pallas_call(
        _mix0_body,
        grid=GRID,
        in_specs=[
            pl.BlockSpec((NC, BR, H), lambda i: (0, i, 0)),
            pl.BlockSpec((NW, BR, 1), lambda i: (0, i, 0)),
            pl.BlockSpec((BR, H), lambda i: (i, 0)),
            pl.BlockSpec((H, H), lambda i: (0, 0)),
            pl.BlockSpec((1, H), lambda i: (0, 0)),
            pl.BlockSpec((H, H), lambda i: (0, 0)),
        ],
        out_specs=pl.BlockSpec((BR, H), lambda i: (i, 0)),
        out_shape=jax.ShapeDtypeStruct((N, H), jnp.float32),
    )(sums, cnts3, h, wl, bl.reshape(1, H), wr)


def _mix1_body(s_ref, c_ref, h_ref, wl_ref, bl_ref, wr_ref, id_ref, o_ref):
    cnt = jnp.sum(c_ref[...], axis=0)
    inv = 1.0 / jnp.maximum(cnt, 1.0)
    mean = (s_ref[0] + s_ref[1]) * inv
    o = _dotT(mean, wl_ref[...]) + bl_ref[...] + _dotT(h_ref[...], wr_ref[...])
    o_ref[...] = jnp.maximum(jnp.maximum(o, 0.0) + id_ref[...], 0.0)


def _mix1(sums, cnts3, h, wl, bl, wr, identity):
    return pl.pallas_call(
        _mix1_body,
        grid=GRID,
        in_specs=[
            pl.BlockSpec((NC, BR, H), lambda i: (0, i, 0)),
            pl.BlockSpec((NW, BR, 1), lambda i: (0, i, 0)),
            pl.BlockSpec((BR, H), lambda i: (i, 0)),
            pl.BlockSpec((H, H), lambda i: (0, 0)),
            pl.BlockSpec((1, H), lambda i: (0, 0)),
            pl.BlockSpec((H, H), lambda i: (0, 0)),
            pl.BlockSpec((BR, H), lambda i: (i, 0)),
        ],
        out_specs=pl.BlockSpec((BR, H), lambda i: (i, 0)),
        out_shape=jax.ShapeDtypeStruct((N, H), jnp.float32),
    )(sums, cnts3, h, wl, bl.reshape(1, H), wr, identity)


def kernel(x, edge_index, W1, b1, Wl0, bl0, Wr0, Wl1, bl1, Wr1):
    src = edge_index[0].astype(jnp.int32)
    dst = edge_index[1].astype(jnp.int32)
    pad = E_PAD - E
    src = jnp.concatenate([src, jnp.zeros((pad,), jnp.int32)])
    dst = jnp.concatenate([dst, jnp.full((pad,), N, jnp.int32)])
    idx = jnp.concatenate([src.reshape(NW, CH, 1, C),
                           dst.reshape(NW, CH, 1, C)], axis=2)
    z2 = jnp.zeros((N_ACC, H), jnp.float32)
    z1 = jnp.zeros((N_ACC,), jnp.float32)

    h0, h = _proj(x, W1, b1)
    sums0, cnts = _agg_cnt_kernel(h, idx, z2, z1)
    cnts3 = cnts.reshape(NW, N_ACC, 1)
    h1 = _mix0(sums0, cnts3, h, Wl0, bl0, Wr0)
    (sums1,) = _agg_kernel(h1, idx, z2)
    out = _mix1(sums1, cnts3, h1, Wl1, bl1, Wr1, h0)
    return out


# speedup vs baseline: 1.2934x; 1.2934x over previous
import jax
import jax.numpy as jnp
from jax import lax
from jax.experimental import pallas as pl
from jax.experimental.pallas import tpu as pltpu
from jax.experimental.pallas import tpu_sc as plsc
import functools

N = 10000
D = 128
H = 128
E = 320000

NC = 2
NS = 16
NW = NC * NS
C = 128
CH = -(-E // (NW * C))
EPT = CH * C
E_PAD = NW * EPT
RPT = 632
N_ACC = NS * RPT

_MESH = plsc.VectorSubcoreMesh(
    core_axis_name="c", subcore_axis_name="s", num_cores=NC, num_subcores=NS)


@functools.partial(
    pl.kernel,
    out_type=[
        jax.ShapeDtypeStruct((NC, N_ACC, H), jnp.float32),
        jax.ShapeDtypeStruct((NC * N_ACC,), jnp.float32),
    ],
    mesh=_MESH,
    scratch_types=[
        pltpu.VMEM((2, C), jnp.int32),
        pltpu.VMEM((C, H), jnp.float32),
        pltpu.VMEM((C,), jnp.float32),
        pltpu.VMEM((RPT,), jnp.float32),
        pltpu.VMEM_SHARED((N_ACC, H), jnp.float32),
        pltpu.VMEM_SHARED((N_ACC,), jnp.float32),
        pltpu.SemaphoreType.DMA,
    ],
)
def _agg_cnt_kernel(h_hbm, idx_hbm, z2_hbm, z1_hbm,
                    sums_out, cnts_out,
                    ij_v, rows_v, ones_v, stage_v, acc_sh, cnt_sh,
                    sem):
    c = lax.axis_index("c")
    s = lax.axis_index("s")
    w = c * NS + s
    r0 = s * RPT
    pltpu.sync_copy(z2_hbm.at[pl.ds(r0, RPT)], acc_sh.at[pl.ds(r0, RPT)])
    pltpu.sync_copy(z1_hbm.at[pl.ds(r0, RPT)], stage_v)
    pltpu.sync_copy(stage_v, cnt_sh.at[pl.ds(r0, RPT)])
    for i in range(C // 16):
        ones_v[pl.ds(i * 16, 16)] = jnp.ones((16,), jnp.float32)
    plsc.subcore_barrier()

    def chunk(j, carry):
        pltpu.sync_copy(idx_hbm.at[w, j], ij_v)
        pltpu.async_copy(h_hbm.at[ij_v.at[0]], rows_v, sem).wait()
        pltpu.sync_copy(rows_v, acc_sh.at[ij_v.at[1]], add=True)
        pltpu.sync_copy(ones_v, cnt_sh.at[ij_v.at[1]], add=True)
        return carry

    lax.fori_loop(0, CH, chunk, 0)
    plsc.subcore_barrier()
    pltpu.sync_copy(acc_sh.at[pl.ds(r0, RPT)], sums_out.at[c, pl.ds(r0, RPT)])
    pltpu.sync_copy(cnt_sh.at[pl.ds(r0, RPT)], stage_v)
    pltpu.sync_copy(stage_v, cnts_out.at[pl.ds(c * N_ACC + r0, RPT)])


@functools.partial(
    pl.kernel,
    out_type=[jax.ShapeDtypeStruct((NC, N_ACC, H), jnp.float32)],
    mesh=_MESH,
    scratch_types=[
        pltpu.VMEM((2, C), jnp.int32),
        pltpu.VMEM((C, H), jnp.float32),
        pltpu.VMEM_SHARED((N_ACC, H), jnp.float32),
        pltpu.SemaphoreType.DMA,
    ],
)
def _agg_kernel(h_hbm, idx_hbm, z2_hbm,
                sums_out, ij_v, rows_v, acc_sh, sem):
    c = lax.axis_index("c")
    s = lax.axis_index("s")
    w = c * NS + s
    r0 = s * RPT
    pltpu.sync_copy(z2_hbm.at[pl.ds(r0, RPT)], acc_sh.at[pl.ds(r0, RPT)])
    plsc.subcore_barrier()

    def chunk(j, carry):
        pltpu.sync_copy(idx_hbm.at[w, j], ij_v)
        pltpu.async_copy(h_hbm.at[ij_v.at[0]], rows_v, sem).wait()
        pltpu.sync_copy(rows_v, acc_sh.at[ij_v.at[1]], add=True)
        return carry

    lax.fori_loop(0, CH, chunk, 0)
    plsc.subcore_barrier()
    pltpu.sync_copy(acc_sh.at[pl.ds(r0, RPT)], sums_out.at[c, pl.ds(r0, RPT)])


BR = 400
GRID = (N // BR,)


def _dotT(a, w):
    return lax.dot_general(a, w, (((1,), (1,)), ((), ())),
                           preferred_element_type=jnp.float32)


def _proj_body(x_ref, w_ref, b_ref, h0_ref, h_ref):
    t = _dotT(x_ref[...], w_ref[...]) + b_ref[...]
    h0_ref[...] = t
    h_ref[...] = jnp.maximum(t, 0.0)


def _proj(x, w1, b1):
    return pl.pallas_call(
        _proj_body,
        grid=GRID,
        in_specs=[
            pl.BlockSpec((BR, D), lambda i: (i, 0)),
            pl.BlockSpec((H, D), lambda i: (0, 0)),
            pl.BlockSpec((1, H), lambda i: (0, 0)),
        ],
        out_specs=[pl.BlockSpec((BR, H), lambda i: (i, 0))] * 2,
        out_shape=[jax.ShapeDtypeStruct((N, H), jnp.float32)] * 2,
    )(x, w1, b1.reshape(1, H))


def _mix0_body(s_ref, c_ref, h_ref, wl_ref, bl_ref, wr_ref, o_ref):
    cnt = c_ref[0] + c_ref[1]
    inv = 1.0 / jnp.maximum(cnt, 1.0)
    mean = (s_ref[0] + s_ref[1]) * inv
    o = _dotT(mean, wl_ref[...]) + bl_ref[...] + _dotT(h_ref[...], wr_ref[...])
    o_ref[...] = jnp.maximum(o, 0.0)


def _mix0(sums, cnts3, h, wl, bl, wr):
    return pl.pallas_call(
        _mix0_body,
        grid=GRID,
        in_specs=[
            pl.BlockSpec((NC, BR, H), lambda i: (0, i, 0)),
            pl.BlockSpec((NC, BR, 1), lambda i: (0, i, 0)),
            pl.BlockSpec((BR, H), lambda i: (i, 0)),
            pl.BlockSpec((H, H), lambda i: (0, 0)),
            pl.BlockSpec((1, H), lambda i: (0, 0)),
            pl.BlockSpec((H, H), lambda i: (0, 0)),
        ],
        out_specs=pl.BlockSpec((BR, H), lambda i: (i, 0)),
        out_shape=jax.ShapeDtypeStruct((N, H), jnp.float32),
    )(sums, cnts3, h, wl, bl.reshape(1, H), wr)


def _mix1_body(s_ref, c_ref, h_ref, wl_ref, bl_ref, wr_ref, id_ref, o_ref):
    cnt = c_ref[0] + c_ref[1]
    inv = 1.0 / jnp.maximum(cnt, 1.0)
    mean = (s_ref[0] + s_ref[1]) * inv
    o = _dotT(mean, wl_ref[...]) + bl_ref[...] + _dotT(h_ref[...], wr_ref[...])
    o_ref[...] = jnp.maximum(jnp.maximum(o, 0.0) + id_ref[...], 0.0)


def _mix1(sums, cnts3, h, wl, bl, wr, identity):
    return pl.pallas_call(
        _mix1_body,
        grid=GRID,
        in_specs=[
            pl.BlockSpec((NC, BR, H), lambda i: (0, i, 0)),
            pl.BlockSpec((NC, BR, 1), lambda i: (0, i, 0)),
            pl.BlockSpec((BR, H), lambda i: (i, 0)),
            pl.BlockSpec((H, H), lambda i: (0, 0)),
            pl.BlockSpec((1, H), lambda i: (0, 0)),
            pl.BlockSpec((H, H), lambda i: (0, 0)),
            pl.BlockSpec((BR, H), lambda i: (i, 0)),
        ],
        out_specs=pl.BlockSpec((BR, H), lambda i: (i, 0)),
        out_shape=jax.ShapeDtypeStruct((N, H), jnp.float32),
    )(sums, cnts3, h, wl, bl.reshape(1, H), wr, identity)


def kernel(x, edge_index, W1, b1, Wl0, bl0, Wr0, Wl1, bl1, Wr1):
    src = edge_index[0].astype(jnp.int32)
    dst = edge_index[1].astype(jnp.int32)
    pad = E_PAD - E
    src = jnp.concatenate([src, jnp.zeros((pad,), jnp.int32)])
    dst = jnp.concatenate([dst, jnp.full((pad,), N, jnp.int32)])
    idx = jnp.concatenate([src.reshape(NW, CH, 1, C),
                           dst.reshape(NW, CH, 1, C)], axis=2)
    z2 = jnp.zeros((N_ACC, H), jnp.float32)
    z1 = jnp.zeros((N_ACC,), jnp.float32)

    h0, h = _proj(x, W1, b1)
    sums0, cnts = _agg_cnt_kernel(h, idx, z2, z1)
    cnts3 = cnts.reshape(NC, N_ACC, 1)
    h1 = _mix0(sums0, cnts3, h, Wl0, bl0, Wr0)
    (sums1,) = _agg_kernel(h1, idx, z2)
    out = _mix1(sums1, cnts3, h1, Wl1, bl1, Wr1, h0)
    return out
